# baseline (device time: 247103 ns/iter reference)
import numpy as np

import jax
import jax.numpy as jnp
from jax import lax
from jax.experimental import pallas as pl
from jax.experimental.pallas import tpu as pltpu

M = 8192
D = 4096
XBLK = M // 2
HALF = XBLK // 2

SIZES = [32, 32, 64] + [128] * 14 + [64, 32, 32]
assert sum(SIZES) == HALF
OFFS = [int(o) for o in np.cumsum([0] + SIZES[:-1])]
NSEG = len(SIZES)
RMAX = max(SIZES)
NSLOT = 4


def kernel(partial, gamma):
    p = partial.reshape(M, D)
    g = gamma.reshape(1, D)

    def body(p_ref, g_ref, out_ref, ystage_ref, sload_v, loc_v, xsend_v,
             xrecv_v, res_v, ysend_v, yin_v, ybuf_v, xs_sems, xr_sems,
             ys_sems, yr_sems, cp_sems):
        my_x = lax.axis_index("x")
        my_y = lax.axis_index("y")
        x_peer = (1 - my_x, my_y)
        y_peer = (my_x, 1 - my_y)

        barrier = pltpu.get_barrier_semaphore()
        for nbr in (x_peer, y_peer):
            pl.semaphore_signal(barrier, inc=1, device_id=nbr,
                                device_id_type=pl.DeviceIdType.MESH)
        pl.semaphore_wait(barrier, 2)

        stage = my_y * HALF
        other = (1 - my_y) * HALF
        send_base = (1 - my_x) * XBLK + stage
        loc_base = my_x * XBLK + stage

        x_rdmas = [None] * NSEG
        y_rdmas = [None] * NSEG
        cp_outs = [None, None]
        cp_youts = [None, None]

        def issue_x(i):
            off, rows = OFFS[i], SIZES[i]
            s = i % NSLOT
            if i >= NSLOT:
                x_rdmas[i - NSLOT].wait_send()
            cp_s = pltpu.make_async_copy(
                p_ref.at[pl.ds(send_base + off, rows), :],
                sload_v.at[pl.ds(0, rows), :], cp_sems.at[0])
            cp_s.start()
            cp_s.wait()
            xsend_v[s, 0:rows, :] = sload_v[0:rows, :].astype(jnp.bfloat16)
            r = pltpu.make_async_remote_copy(
                src_ref=xsend_v.at[s, pl.ds(0, rows), :],
                dst_ref=xrecv_v.at[i, pl.ds(0, rows), :],
                send_sem=xs_sems.at[s],
                recv_sem=xr_sems.at[i],
                device_id=x_peer,
                device_id_type=pl.DeviceIdType.MESH,
            )
            r.start()
            x_rdmas[i] = r

        def process_x(j):
            off, rows = OFFS[j], SIZES[j]
            b = j % 2
            s = j % NSLOT
            cp_l = pltpu.make_async_copy(
                p_ref.at[pl.ds(loc_base + off, rows), :],
                loc_v.at[pl.ds(0, rows), :], cp_sems.at[1])
            cp_l.start()
            if j >= NSLOT:
                y_rdmas[j - NSLOT].wait_send()
            if j >= 2:
                cp_outs[b].wait()
            x_rdmas[j].wait_recv()
            cp_l.wait()
            y = loc_v[0:rows, :] + xrecv_v[j, 0:rows, :].astype(jnp.float32)
            ms = jnp.mean(y * y, axis=-1, keepdims=True)
            r = y * lax.rsqrt(ms + 1e-6) * g_ref[...]
            res_v[b, 0:rows, :] = r
            ysend_v[s, 0:rows, :] = r.astype(jnp.bfloat16)
            ry = pltpu.make_async_remote_copy(
                src_ref=ysend_v.at[s, pl.ds(0, rows), :],
                dst_ref=ystage_ref.at[pl.ds(off, rows), :],
                send_sem=ys_sems.at[s],
                recv_sem=yr_sems.at[j],
                device_id=y_peer,
                device_id_type=pl.DeviceIdType.MESH,
            )
            ry.start()
            y_rdmas[j] = ry
            cp_outs[b] = pltpu.make_async_copy(
                res_v.at[b, pl.ds(0, rows), :],
                out_ref.at[pl.ds(stage + off, rows), :], cp_sems.at[2 + b])
            cp_outs[b].start()

        def process_y(k):
            off, rows = OFFS[k], SIZES[k]
            c = k % 2
            y_rdmas[k].wait_recv()
            if k >= 2:
                cp_youts[c].wait()
            cp_in = pltpu.make_async_copy(
                ystage_ref.at[pl.ds(off, rows), :],
                yin_v.at[c, pl.ds(0, rows), :], cp_sems.at[4 + c])
            cp_in.start()
            cp_in.wait()
            ybuf_v[c, 0:rows, :] = yin_v[c, 0:rows, :].astype(jnp.float32)
            cp_youts[c] = pltpu.make_async_copy(
                ybuf_v.at[c, pl.ds(0, rows), :],
                out_ref.at[pl.ds(other + off, rows), :], cp_sems.at[6 + c])
            cp_youts[c].start()

        for i in range(NSEG):
            issue_x(i)
            if i >= 1:
                process_x(i - 1)
            if i >= 3:
                process_y(i - 3)
        process_x(NSEG - 1)
        for k in range(NSEG - 3, NSEG):
            process_y(k)

        for i in range(NSEG - NSLOT, NSEG):
            x_rdmas[i].wait_send()
            y_rdmas[i].wait_send()
        cp_outs[0].wait()
        cp_outs[1].wait()
        cp_youts[0].wait()
        cp_youts[1].wait()

    out, _ = pl.pallas_call(
        body,
        out_shape=(
            jax.ShapeDtypeStruct((XBLK, D), jnp.float32),
            jax.ShapeDtypeStruct((HALF, D), jnp.bfloat16),
        ),
        in_specs=[
            pl.BlockSpec(memory_space=pl.ANY),
            pl.BlockSpec(memory_space=pltpu.MemorySpace.VMEM),
        ],
        out_specs=(
            pl.BlockSpec(memory_space=pl.ANY),
            pl.BlockSpec(memory_space=pl.ANY),
        ),
        scratch_shapes=[
            pltpu.VMEM((RMAX, D), jnp.float32),
            pltpu.VMEM((RMAX, D), jnp.float32),
            pltpu.VMEM((NSLOT, RMAX, D), jnp.bfloat16),
            pltpu.VMEM((NSEG, RMAX, D), jnp.bfloat16),
            pltpu.VMEM((2, RMAX, D), jnp.float32),
            pltpu.VMEM((NSLOT, RMAX, D), jnp.bfloat16),
            pltpu.VMEM((2, RMAX, D), jnp.bfloat16),
            pltpu.VMEM((2, RMAX, D), jnp.float32),
            pltpu.SemaphoreType.DMA((NSLOT,)),
            pltpu.SemaphoreType.DMA((NSEG,)),
            pltpu.SemaphoreType.DMA((NSLOT,)),
            pltpu.SemaphoreType.DMA((NSEG,)),
            pltpu.SemaphoreType.DMA((8,)),
        ],
        compiler_params=pltpu.CompilerParams(
            collective_id=0, vmem_limit_bytes=56 * 1024 * 1024),
    )(p, g)
    return out


# device time: 246435 ns/iter; 1.0027x vs baseline; 1.0027x over previous
import numpy as np

import jax
import jax.numpy as jnp
from jax import lax
from jax.experimental import pallas as pl
from jax.experimental.pallas import tpu as pltpu

M = 8192
D = 4096
XBLK = M // 2
HALF = XBLK // 2

SIZES = [64, 64] + [128] * 14 + [64, 64]
assert sum(SIZES) == HALF
OFFS = [int(o) for o in np.cumsum([0] + SIZES[:-1])]
NSEG = len(SIZES)
RMAX = max(SIZES)
NSLOT = 4


def kernel(partial, gamma):
    p = partial.reshape(M, D)
    g = gamma.reshape(1, D)

    def body(p_ref, g_ref, out_ref, ystage_ref, sload_v, loc_v, xsend_v,
             xrecv_v, res_v, ysend_v, yin_v, ybuf_v, xs_sems, xr_sems,
             ys_sems, yr_sems, cp_sems):
        my_x = lax.axis_index("x")
        my_y = lax.axis_index("y")
        x_peer = (1 - my_x, my_y)
        y_peer = (my_x, 1 - my_y)

        barrier = pltpu.get_barrier_semaphore()
        for nbr in (x_peer, y_peer):
            pl.semaphore_signal(barrier, inc=1, device_id=nbr,
                                device_id_type=pl.DeviceIdType.MESH)
        pl.semaphore_wait(barrier, 2)

        stage = my_y * HALF
        other = (1 - my_y) * HALF
        send_base = (1 - my_x) * XBLK + stage
        loc_base = my_x * XBLK + stage

        x_rdmas = [None] * NSEG
        y_rdmas = [None] * NSEG
        cp_outs = [None, None]
        cp_youts = [None, None]

        def issue_x(i):
            off, rows = OFFS[i], SIZES[i]
            s = i % NSLOT
            if i >= NSLOT:
                x_rdmas[i - NSLOT].wait_send()
            cp_s = pltpu.make_async_copy(
                p_ref.at[pl.ds(send_base + off, rows), :],
                sload_v.at[pl.ds(0, rows), :], cp_sems.at[0])
            cp_s.start()
            cp_s.wait()
            xsend_v[s, 0:rows, :] = sload_v[0:rows, :].astype(jnp.bfloat16)
            r = pltpu.make_async_remote_copy(
                src_ref=xsend_v.at[s, pl.ds(0, rows), :],
                dst_ref=xrecv_v.at[i, pl.ds(0, rows), :],
                send_sem=xs_sems.at[s],
                recv_sem=xr_sems.at[i],
                device_id=x_peer,
                device_id_type=pl.DeviceIdType.MESH,
            )
            r.start()
            x_rdmas[i] = r

        def process_x(j):
            off, rows = OFFS[j], SIZES[j]
            b = j % 2
            s = j % NSLOT
            cp_l = pltpu.make_async_copy(
                p_ref.at[pl.ds(loc_base + off, rows), :],
                loc_v.at[pl.ds(0, rows), :], cp_sems.at[1])
            cp_l.start()
            if j >= NSLOT:
                y_rdmas[j - NSLOT].wait_send()
            if j >= 2:
                cp_outs[b].wait()
            x_rdmas[j].wait_recv()
            cp_l.wait()
            y = loc_v[0:rows, :] + xrecv_v[j, 0:rows, :].astype(jnp.float32)
            ms = jnp.mean(y * y, axis=-1, keepdims=True)
            r = y * lax.rsqrt(ms + 1e-6) * g_ref[...]
            res_v[b, 0:rows, :] = r
            ysend_v[s, 0:rows, :] = r.astype(jnp.bfloat16)
            ry = pltpu.make_async_remote_copy(
                src_ref=ysend_v.at[s, pl.ds(0, rows), :],
                dst_ref=ystage_ref.at[pl.ds(off, rows), :],
                send_sem=ys_sems.at[s],
                recv_sem=yr_sems.at[j],
                device_id=y_peer,
                device_id_type=pl.DeviceIdType.MESH,
            )
            ry.start()
            y_rdmas[j] = ry
            cp_outs[b] = pltpu.make_async_copy(
                res_v.at[b, pl.ds(0, rows), :],
                out_ref.at[pl.ds(stage + off, rows), :], cp_sems.at[2 + b])
            cp_outs[b].start()

        def process_y(k):
            off, rows = OFFS[k], SIZES[k]
            c = k % 2
            y_rdmas[k].wait_recv()
            if k >= 2:
                cp_youts[c].wait()
            cp_in = pltpu.make_async_copy(
                ystage_ref.at[pl.ds(off, rows), :],
                yin_v.at[c, pl.ds(0, rows), :], cp_sems.at[4 + c])
            cp_in.start()
            cp_in.wait()
            ybuf_v[c, 0:rows, :] = yin_v[c, 0:rows, :].astype(jnp.float32)
            cp_youts[c] = pltpu.make_async_copy(
                ybuf_v.at[c, pl.ds(0, rows), :],
                out_ref.at[pl.ds(other + off, rows), :], cp_sems.at[6 + c])
            cp_youts[c].start()

        for i in range(NSEG):
            issue_x(i)
            if i >= 1:
                process_x(i - 1)
            if i >= 3:
                process_y(i - 3)
        process_x(NSEG - 1)
        for k in range(NSEG - 3, NSEG):
            process_y(k)

        for i in range(NSEG - NSLOT, NSEG):
            x_rdmas[i].wait_send()
            y_rdmas[i].wait_send()
        cp_outs[0].wait()
        cp_outs[1].wait()
        cp_youts[0].wait()
        cp_youts[1].wait()

    out, _ = pl.pallas_call(
        body,
        out_shape=(
            jax.ShapeDtypeStruct((XBLK, D), jnp.float32),
            jax.ShapeDtypeStruct((HALF, D), jnp.bfloat16),
        ),
        in_specs=[
            pl.BlockSpec(memory_space=pl.ANY),
            pl.BlockSpec(memory_space=pltpu.MemorySpace.VMEM),
        ],
        out_specs=(
            pl.BlockSpec(memory_space=pl.ANY),
            pl.BlockSpec(memory_space=pl.ANY),
        ),
        scratch_shapes=[
            pltpu.VMEM((RMAX, D), jnp.float32),
            pltpu.VMEM((RMAX, D), jnp.float32),
            pltpu.VMEM((NSLOT, RMAX, D), jnp.bfloat16),
            pltpu.VMEM((NSEG, RMAX, D), jnp.bfloat16),
            pltpu.VMEM((2, RMAX, D), jnp.float32),
            pltpu.VMEM((NSLOT, RMAX, D), jnp.bfloat16),
            pltpu.VMEM((2, RMAX, D), jnp.bfloat16),
            pltpu.VMEM((2, RMAX, D), jnp.float32),
            pltpu.SemaphoreType.DMA((NSLOT,)),
            pltpu.SemaphoreType.DMA((NSEG,)),
            pltpu.SemaphoreType.DMA((NSLOT,)),
            pltpu.SemaphoreType.DMA((NSEG,)),
            pltpu.SemaphoreType.DMA((8,)),
        ],
        compiler_params=pltpu.CompilerParams(
            collective_id=0, vmem_limit_bytes=56 * 1024 * 1024),
    )(p, g)
    return out
